# baseline (speedup 1.0000x reference)
import jax
import jax.numpy as jnp
from jax import lax
from jax.experimental import pallas as pl
from jax.experimental.pallas import tpu as pltpu
from jax.experimental.pallas import tpu_sc as plsc

E = 3_200_000
N = 100_000
D = 4
H = 4
L = 16
NC = 2
NS = 16
NW = NC * NS
C = 1024
IB = 128
RPC = C // IB
RW = 8
NCHUNKS = E // C
BASE_CHUNKS = NCHUNKS // NW
EXTRA = NCHUNKS % NW
NP = 100_096
RPT = NP // NS

_mesh = plsc.VectorSubcoreMesh(core_axis_name="c", subcore_axis_name="s")


def _worker_id():
    return lax.axis_index("s") * NC + lax.axis_index("c")


def _att_lanes(att_v):
    return [[att_v[h * D + d] for d in range(D)] for h in range(H)]


def _logits(ef_v, row, att_vec):
    cols = [plsc.load_gather(ef_v, [row, jnp.full((L,), d, jnp.int32)])
            for d in range(D)]
    ps = []
    for h in range(H):
        e = att_vec[h][0] * cols[0]
        for d in range(1, D):
            e = e + att_vec[h][d] * cols[d]
        ps.append(jnp.exp(e))
    return cols, ps


def _sum_body(ef_hbm, att_hbm, dst_hbm, z_hbm, s_part, att_v, ef_v, dst_v,
              p_v, s_sh):
    cid = lax.axis_index("c")
    sid = lax.axis_index("s")
    wid = _worker_id()
    pltpu.sync_copy(z_hbm.at[pl.ds(sid * RPT, RPT)],
                    s_sh.at[pl.ds(sid * RPT, RPT)])
    pltpu.sync_copy(att_hbm, att_v)
    plsc.subcore_barrier()

    att_vec = _att_lanes(att_v)
    iota = lax.iota(jnp.int32, L)
    nch = BASE_CHUNKS + jnp.where(wid < EXTRA, 1, 0)

    def chunk(i, carry):
        ch = wid + i * NW
        pltpu.sync_copy(ef_hbm.at[pl.ds(ch * C, C)], ef_v)
        pltpu.sync_copy(dst_hbm.at[1, pl.ds(ch * RPC, RPC)], dst_v)

        def grp(g, c2):
            row = iota + g * L
            _, ps = _logits(ef_v, row, att_vec)
            for h in range(H):
                plsc.store_scatter(p_v, [row, jnp.full((L,), h, jnp.int32)],
                                   ps[h])
            return c2

        lax.fori_loop(0, C // L, grp, 0)
        for j in range(RPC):
            pltpu.sync_copy(p_v.at[pl.ds(j * IB, IB)],
                            s_sh.at[dst_v.at[j]], add=True)
        return carry

    lax.fori_loop(0, nch, chunk, 0)
    plsc.subcore_barrier()
    pltpu.sync_copy(s_sh.at[pl.ds(sid * RPT, RPT)],
                    s_part.at[cid, pl.ds(sid * RPT, RPT)])


def _out_body(ef_hbm, att_hbm, dst_hbm, z_hbm, s_hbm, out_part, att_v, ef_v,
              dst_v, srow_v, msg_v, s_sh, acc_sh):
    cid = lax.axis_index("c")
    sid = lax.axis_index("s")
    wid = _worker_id()
    pltpu.sync_copy(s_hbm.at[pl.ds(sid * RPT, RPT)],
                    s_sh.at[pl.ds(sid * RPT, RPT)])
    pltpu.sync_copy(z_hbm.at[pl.ds(sid * RPT, RPT)],
                    acc_sh.at[pl.ds(sid * RPT, RPT)])
    pltpu.sync_copy(att_hbm, att_v)
    plsc.subcore_barrier()

    att_vec = _att_lanes(att_v)
    iota = lax.iota(jnp.int32, L)
    nch = BASE_CHUNKS + jnp.where(wid < EXTRA, 1, 0)
    quarter = jnp.float32(1.0 / H)

    def chunk(i, carry):
        ch = wid + i * NW
        pltpu.sync_copy(ef_hbm.at[pl.ds(ch * C, C)], ef_v)
        pltpu.sync_copy(dst_hbm.at[1, pl.ds(ch * RPC, RPC)], dst_v)
        for j in range(RPC):
            pltpu.sync_copy(s_sh.at[dst_v.at[j]],
                            srow_v.at[pl.ds(j * IB, IB)])

        def grp(g, c2):
            row = iota + g * L
            cols, ps = _logits(ef_v, row, att_vec)
            w = jnp.zeros((L,), jnp.float32)
            for h in range(H):
                sh = plsc.load_gather(srow_v,
                                      [row, jnp.full((L,), h, jnp.int32)])
                w = w + ps[h] / sh
            w = w * quarter
            for d in range(D):
                plsc.store_scatter(msg_v, [row, jnp.full((L,), d, jnp.int32)],
                                   cols[d] * w)
            return c2

        lax.fori_loop(0, C // L, grp, 0)
        for j in range(RPC):
            pltpu.sync_copy(msg_v.at[pl.ds(j * IB, IB)],
                            acc_sh.at[dst_v.at[j]], add=True)
        return carry

    lax.fori_loop(0, nch, chunk, 0)
    plsc.subcore_barrier()
    pltpu.sync_copy(acc_sh.at[pl.ds(sid * RPT, RPT)],
                    out_part.at[cid, pl.ds(sid * RPT, RPT)])


_sc_params = pltpu.CompilerParams(needs_layout_passes=False,
                                  use_tc_tiling_on_sc=False)

_sum_kernel = pl.kernel(
    _sum_body,
    out_type=jax.ShapeDtypeStruct((NC, NP, RW), jnp.float32),
    mesh=_mesh,
    compiler_params=_sc_params,
    scratch_types=[
        pltpu.VMEM((H * D, L), jnp.float32),
        pltpu.VMEM((C, D), jnp.float32),
        pltpu.VMEM((RPC, IB), jnp.int32),
        pltpu.VMEM((C, RW), jnp.float32),
        pltpu.VMEM_SHARED((NP, RW), jnp.float32),
    ],
)

_out_kernel = pl.kernel(
    _out_body,
    out_type=jax.ShapeDtypeStruct((NC, NP, RW), jnp.float32),
    mesh=_mesh,
    compiler_params=_sc_params,
    scratch_types=[
        pltpu.VMEM((H * D, L), jnp.float32),
        pltpu.VMEM((C, D), jnp.float32),
        pltpu.VMEM((RPC, IB), jnp.int32),
        pltpu.VMEM((C, RW), jnp.float32),
        pltpu.VMEM((C, RW), jnp.float32),
        pltpu.VMEM_SHARED((NP, RW), jnp.float32),
        pltpu.VMEM_SHARED((NP, RW), jnp.float32),
    ],
)


def _add_halves_body(a_ref, o_ref):
    o_ref[...] = a_ref[0] + a_ref[1]


def _combine(part):
    rows = (NP * RW) // 128
    x = part.reshape(NC, rows, 128)
    y = pl.pallas_call(
        _add_halves_body,
        out_shape=jax.ShapeDtypeStruct((rows, 128), jnp.float32),
    )(x)
    return y.reshape(NP, RW)


@jax.jit
def kernel(edge_feat, att, edge_index):
    dst3d = edge_index.reshape(2, E // IB, IB)
    att16 = jnp.tile(att.reshape(H * D, 1), (1, L))
    z = jnp.zeros((NP, RW), jnp.float32)
    s_part = _sum_kernel(edge_feat, att16, dst3d, z)
    s = _combine(s_part)
    out_part = _out_kernel(edge_feat, att16, dst3d, z, s)
    return _combine(out_part)[:N, :D]


# speedup vs baseline: 1.1567x; 1.1567x over previous
import jax
import jax.numpy as jnp
from jax import lax
from jax.experimental import pallas as pl
from jax.experimental.pallas import tpu as pltpu
from jax.experimental.pallas import tpu_sc as plsc

E = 3_200_000
N = 100_000
D = 4
H = 4
L = 16
NC = 2
NS = 16
NW = NC * NS
C = 1024
IB = 128
RPC = C // IB
RW = 8
NCHUNKS = E // C
BASE_CHUNKS = NCHUNKS // NW
EXTRA = NCHUNKS % NW
NP = 100_096
RPT = NP // NS

_mesh = plsc.VectorSubcoreMesh(core_axis_name="c", subcore_axis_name="s")


def _worker_id():
    return lax.axis_index("s") * NC + lax.axis_index("c")


def _att_lanes(att_v):
    return [[att_v[h * D + d] for d in range(D)] for h in range(H)]


def _logits(ef_v, row, att_vec):
    base = row * D
    cols = [plsc.load_gather(ef_v, [base + d]) for d in range(D)]
    ps = []
    for h in range(H):
        e = att_vec[h][0] * cols[0]
        for d in range(1, D):
            e = e + att_vec[h][d] * cols[d]
        ps.append(jnp.exp(e))
    return cols, ps


def _sum_body(ef_hbm, att_hbm, dst_hbm, z_hbm, s_part, att_v, ef_v, dst_v,
              p_v, s_sh):
    cid = lax.axis_index("c")
    sid = lax.axis_index("s")
    wid = _worker_id()
    pltpu.sync_copy(z_hbm.at[pl.ds(sid * RPT, RPT)],
                    s_sh.at[pl.ds(sid * RPT, RPT)])
    pltpu.sync_copy(att_hbm, att_v)
    plsc.subcore_barrier()

    att_vec = _att_lanes(att_v)
    iota = lax.iota(jnp.int32, L)
    nch = BASE_CHUNKS + jnp.where(wid < EXTRA, 1, 0)

    def chunk(i, carry):
        ch = wid + i * NW
        pltpu.sync_copy(ef_hbm.at[pl.ds(ch * C * D, C * D)], ef_v)
        pltpu.sync_copy(dst_hbm.at[1, pl.ds(ch * RPC, RPC)], dst_v)

        def grp(g, c2):
            row = iota + g * L
            _, ps = _logits(ef_v, row, att_vec)
            for h in range(H):
                plsc.store_scatter(p_v, [row, jnp.full((L,), h, jnp.int32)],
                                   ps[h])
            return c2

        lax.fori_loop(0, C // L, grp, 0)
        for j in range(RPC):
            pltpu.sync_copy(p_v.at[pl.ds(j * IB, IB)],
                            s_sh.at[dst_v.at[j]], add=True)
        return carry

    lax.fori_loop(0, nch, chunk, 0)
    plsc.subcore_barrier()
    pltpu.sync_copy(s_sh.at[pl.ds(sid * RPT, RPT)],
                    s_part.at[cid, pl.ds(sid * RPT, RPT)])


def _out_body(ef_hbm, att_hbm, dst_hbm, z_hbm, s_hbm, out_part, att_v, ef_v,
              dst_v, srow_v, msg_v, s_sh, acc_sh):
    cid = lax.axis_index("c")
    sid = lax.axis_index("s")
    wid = _worker_id()
    pltpu.sync_copy(s_hbm.at[pl.ds(sid * RPT, RPT)],
                    s_sh.at[pl.ds(sid * RPT, RPT)])
    pltpu.sync_copy(z_hbm.at[pl.ds(sid * RPT, RPT)],
                    acc_sh.at[pl.ds(sid * RPT, RPT)])
    pltpu.sync_copy(att_hbm, att_v)
    plsc.subcore_barrier()

    att_vec = _att_lanes(att_v)
    iota = lax.iota(jnp.int32, L)
    nch = BASE_CHUNKS + jnp.where(wid < EXTRA, 1, 0)
    quarter = jnp.float32(1.0 / H)

    def chunk(i, carry):
        ch = wid + i * NW
        pltpu.sync_copy(ef_hbm.at[pl.ds(ch * C * D, C * D)], ef_v)
        pltpu.sync_copy(dst_hbm.at[1, pl.ds(ch * RPC, RPC)], dst_v)
        for j in range(RPC):
            pltpu.sync_copy(s_sh.at[dst_v.at[j]],
                            srow_v.at[pl.ds(j * IB, IB)])

        def grp(g, c2):
            row = iota + g * L
            cols, ps = _logits(ef_v, row, att_vec)
            w = jnp.zeros((L,), jnp.float32)
            for h in range(H):
                sh = plsc.load_gather(srow_v,
                                      [row, jnp.full((L,), h, jnp.int32)])
                w = w + ps[h] / sh
            w = w * quarter
            for d in range(D):
                plsc.store_scatter(msg_v, [row, jnp.full((L,), d, jnp.int32)],
                                   cols[d] * w)
            return c2

        lax.fori_loop(0, C // L, grp, 0)
        for j in range(RPC):
            pltpu.sync_copy(msg_v.at[pl.ds(j * IB, IB)],
                            acc_sh.at[dst_v.at[j]], add=True)
        return carry

    lax.fori_loop(0, nch, chunk, 0)
    plsc.subcore_barrier()
    pltpu.sync_copy(acc_sh.at[pl.ds(sid * RPT, RPT)],
                    out_part.at[cid, pl.ds(sid * RPT, RPT)])


_sc_params = pltpu.CompilerParams(needs_layout_passes=False,
                                  use_tc_tiling_on_sc=False)

_sum_kernel = pl.kernel(
    _sum_body,
    out_type=jax.ShapeDtypeStruct((NC, NP, RW), jnp.float32),
    mesh=_mesh,
    compiler_params=_sc_params,
    scratch_types=[
        pltpu.VMEM((H * D, L), jnp.float32),
        pltpu.VMEM((C * D,), jnp.float32),
        pltpu.VMEM((RPC, IB), jnp.int32),
        pltpu.VMEM((C, RW), jnp.float32),
        pltpu.VMEM_SHARED((NP, RW), jnp.float32),
    ],
)

_out_kernel = pl.kernel(
    _out_body,
    out_type=jax.ShapeDtypeStruct((NC, NP, RW), jnp.float32),
    mesh=_mesh,
    compiler_params=_sc_params,
    scratch_types=[
        pltpu.VMEM((H * D, L), jnp.float32),
        pltpu.VMEM((C * D,), jnp.float32),
        pltpu.VMEM((RPC, IB), jnp.int32),
        pltpu.VMEM((C, RW), jnp.float32),
        pltpu.VMEM((C, RW), jnp.float32),
        pltpu.VMEM_SHARED((NP, RW), jnp.float32),
        pltpu.VMEM_SHARED((NP, RW), jnp.float32),
    ],
)


def _add_halves_body(a_ref, o_ref):
    o_ref[...] = a_ref[0] + a_ref[1]


def _combine(part):
    rows = (NP * RW) // 128
    x = part.reshape(NC, rows, 128)
    y = pl.pallas_call(
        _add_halves_body,
        out_shape=jax.ShapeDtypeStruct((rows, 128), jnp.float32),
    )(x)
    return y.reshape(NP, RW)


@jax.jit
def kernel(edge_feat, att, edge_index):
    ef1 = edge_feat.reshape(E * D)
    dst3d = edge_index.reshape(2, E // IB, IB)
    att16 = jnp.tile(att.reshape(H * D, 1), (1, L))
    z = jnp.zeros((NP, RW), jnp.float32)
    s_part = _sum_kernel(ef1, att16, dst3d, z)
    s = _combine(s_part)
    out_part = _out_kernel(ef1, att16, dst3d, z, s)
    return _combine(out_part)[:N, :D]
